# baseline (speedup 1.0000x reference)
import functools

import jax
import jax.numpy as jnp
from jax import lax
from jax.experimental import pallas as pl
from jax.experimental.pallas import tpu as pltpu
from jax.experimental.pallas import tpu_sc as plsc

F32 = jnp.float32
_N, _H, _MD, _NP, _RH = 8192, 2048, 512, 64, 2048
_SCHEMA, _MAXHOP, _SRCVOC = 2048, 8, 8
_RS = 0.2
_BN = 256
_NB = _N // _BN
_BNA = 128
_NBA = _N // _BNA
_BJ = 256
_SC_C = 64
_SQRT1_2 = 0.7071067811865476


def _gelu_exact(x):
    return 0.5 * x * (1.0 + lax.erf(x * _SQRT1_2))


def _layernorm(x, g, b, eps=1e-5):
    mu = jnp.mean(x, axis=-1, keepdims=True)
    xc = x - mu
    var = jnp.mean(xc * xc, axis=-1, keepdims=True)
    return xc * lax.rsqrt(var + eps) * g + b


def _sc_gather(table, idx):
    info = plsc.get_sparse_core_info()
    nw = info.num_cores * info.num_subcores
    rows_per_w = _N // nw
    n_chunks = rows_per_w // _SC_C
    mesh = plsc.VectorSubcoreMesh(core_axis_name="c", subcore_axis_name="s")
    md = table.shape[1]

    @functools.partial(
        pl.kernel, mesh=mesh,
        out_type=jax.ShapeDtypeStruct((_N, md), F32),
        scratch_types=[pltpu.VMEM((rows_per_w,), jnp.int32),
                       pltpu.VMEM((_SC_C, md), F32),
                       pltpu.VMEM((_SC_C, md), F32),
                       pltpu.SemaphoreType.DMA],
    )
    def gather_k(table_hbm, idx_hbm, out_hbm, idx_v, buf0, buf1, sem):
        wid = lax.axis_index("s") * info.num_cores + lax.axis_index("c")
        base = wid * rows_per_w
        pltpu.sync_copy(idx_hbm.at[pl.ds(base, rows_per_w)], idx_v)
        bufs = (buf0, buf1)
        cps = [
            pltpu.async_copy(table_hbm.at[idx_v.at[pl.ds(k * _SC_C, _SC_C)]],
                             bufs[k], sem)
            for k in range(2)
        ]
        for k in range(n_chunks):
            cps[k % 2].wait()
            pltpu.sync_copy(bufs[k % 2],
                            out_hbm.at[pl.ds(base + k * _SC_C, _SC_C)])
            if k + 2 < n_chunks:
                cps[k % 2] = pltpu.async_copy(
                    table_hbm.at[idx_v.at[pl.ds((k + 2) * _SC_C, _SC_C)]],
                    bufs[k % 2], sem)

    return gather_k(table, idx)


def _stage_ab_body(pair_ref, ep_ref, mds_ref, hid_ref, pid_ref,
                   w_pair_ref, w_ep_ref, w_s_ref, w_h_ref, w_p_ref,
                   hop_ref, src_ref, b1_ref, g1_ref, be1_ref,
                   w2_ref, b2_ref, proto_ref, ctx_ref, pw_ref):
    acc = jnp.dot(pair_ref[...], w_pair_ref[...], preferred_element_type=F32)
    acc = acc + jnp.dot(ep_ref[...], w_ep_ref[...], preferred_element_type=F32)
    acc = acc + jnp.dot(mds_ref[...], w_s_ref[...], preferred_element_type=F32)
    th = jnp.dot(hop_ref[...], w_h_ref[...], preferred_element_type=F32)
    tp = jnp.dot(src_ref[...], w_p_ref[...], preferred_element_type=F32)
    hid = hid_ref[0, 0, :]
    pid = pid_ref[0, 0, :]
    oh_h = jnp.where(hid[:, None] == lax.broadcasted_iota(jnp.int32, (_BNA, 16), 1),
                     1.0, 0.0).astype(F32)
    oh_p = jnp.where(pid[:, None] == lax.broadcasted_iota(jnp.int32, (_BNA, 8), 1),
                     1.0, 0.0).astype(F32)
    acc = acc + jnp.dot(oh_h, th, preferred_element_type=F32,
                        precision=lax.Precision.HIGHEST)
    acc = acc + jnp.dot(oh_p, tp, preferred_element_type=F32,
                        precision=lax.Precision.HIGHEST)
    acc = acc + b1_ref[...]
    h = _layernorm(acc, g1_ref[...], be1_ref[...])
    h = _gelu_exact(h)
    logits = jnp.dot(h, w2_ref[...], preferred_element_type=F32) + b2_ref[...]
    cur = logits
    t = None
    for _ in range(4):
        t = jnp.max(cur, axis=-1, keepdims=True)
        cur = jnp.where(cur >= t, -jnp.inf, cur)
    mask = logits >= t
    m1 = jnp.max(logits, axis=-1, keepdims=True)
    e = jnp.exp(logits - m1)
    es = jnp.where(mask, e, 0.0)
    pw = es / jnp.sum(es, axis=-1, keepdims=True)
    pw_ref[...] = pw
    ctx_ref[...] = jnp.dot(pw, proto_ref[...], preferred_element_type=F32)


def _stage_ab(pair, ep, mds, hid3, pid3, w_pair, w_ep, w_s, w_h, w_p,
              hop_pad, src, b1, g1, be1, w2, b2, proto):
    return pl.pallas_call(
        _stage_ab_body,
        grid=(_NBA,),
        in_specs=[
            pl.BlockSpec((_BNA, _H), lambda n: (n, 0)),
            pl.BlockSpec((_BNA, _H), lambda n: (n, 0)),
            pl.BlockSpec((_BNA, _MD), lambda n: (n, 0)),
            pl.BlockSpec((1, 1, _BNA), lambda n: (n, 0, 0)),
            pl.BlockSpec((1, 1, _BNA), lambda n: (n, 0, 0)),
            pl.BlockSpec((_H, _H), lambda n: (0, 0)),
            pl.BlockSpec((_H, _H), lambda n: (0, 0)),
            pl.BlockSpec((_MD, _H), lambda n: (0, 0)),
            pl.BlockSpec((_MD, _H), lambda n: (0, 0)),
            pl.BlockSpec((_MD, _H), lambda n: (0, 0)),
            pl.BlockSpec((16, _MD), lambda n: (0, 0)),
            pl.BlockSpec((8, _MD), lambda n: (0, 0)),
            pl.BlockSpec((_H,), lambda n: (0,)),
            pl.BlockSpec((_H,), lambda n: (0,)),
            pl.BlockSpec((_H,), lambda n: (0,)),
            pl.BlockSpec((_H, _NP), lambda n: (0, 0)),
            pl.BlockSpec((_NP,), lambda n: (0,)),
            pl.BlockSpec((_NP, _H), lambda n: (0, 0)),
        ],
        out_specs=(pl.BlockSpec((_BNA, _H), lambda n: (n, 0)),
                   pl.BlockSpec((_BNA, _NP), lambda n: (n, 0))),
        out_shape=(jax.ShapeDtypeStruct((_N, _H), F32),
                   jax.ShapeDtypeStruct((_N, _NP), F32)),
    )(pair, ep, mds, hid3, pid3, w_pair, w_ep, w_s, w_h, w_p, hop_pad, src,
      b1, g1, be1, w2, b2, proto)


def _mm1_body(ep_ref, ctx_ref, wmu_ref, wmg_ref, d_ref, g_ref):
    ep = ep_ref[...]
    ctx = ctx_ref[...]
    f2 = jnp.abs(ep - ctx)
    f3 = ep * ctx
    for w_ref, o_ref in ((wmu_ref, d_ref), (wmg_ref, g_ref)):
        acc = jnp.dot(ep, w_ref[0:_H, :], preferred_element_type=F32)
        acc = acc + jnp.dot(ctx, w_ref[_H:2 * _H, :],
                            preferred_element_type=F32)
        acc = acc + jnp.dot(f2, w_ref[2 * _H:3 * _H, :],
                            preferred_element_type=F32)
        acc = acc + jnp.dot(f3, w_ref[3 * _H:4 * _H, :],
                            preferred_element_type=F32)
        o_ref[...] = acc


def _mm1(ep, ctx, wmu, wmg):
    nj = _H // _BJ
    return pl.pallas_call(
        _mm1_body,
        grid=(nj, _NB),
        in_specs=[
            pl.BlockSpec((_BN, _H), lambda j, n: (n, 0)),
            pl.BlockSpec((_BN, _H), lambda j, n: (n, 0)),
            pl.BlockSpec((4 * _H, _BJ), lambda j, n: (0, j)),
            pl.BlockSpec((4 * _H, _BJ), lambda j, n: (0, j)),
        ],
        out_specs=(pl.BlockSpec((_BN, _BJ), lambda j, n: (n, j)),
                   pl.BlockSpec((_BN, _BJ), lambda j, n: (n, j))),
        out_shape=(jax.ShapeDtypeStruct((_N, _H), F32),
                   jax.ShapeDtypeStruct((_N, _H), F32)),
    )(ep, ctx, wmu, wmg)


def _stage_c_body(d1_ref, g1_ref, ep_ref, mu_b1_ref, mu_g_ref, mu_be_ref,
                  mu_w2_ref, mu_b2_ref, mg_b1_ref, mg_w2_ref, mg_b2_ref,
                  n_g_ref, n_be_ref, out_ref):
    d1 = d1_ref[...] + mu_b1_ref[...]
    d1 = _layernorm(d1, mu_g_ref[...], mu_be_ref[...])
    d1 = _gelu_exact(d1)
    d = jnp.dot(d1, mu_w2_ref[...], preferred_element_type=F32) + mu_b2_ref[...]
    g1 = g1_ref[...] + mg_b1_ref[...]
    g1 = _gelu_exact(g1)
    g = jnp.dot(g1, mg_w2_ref[...], preferred_element_type=F32) + mg_b2_ref[...]
    g = jax.nn.sigmoid(g)
    u = ep_ref[...] + _RS * g * d
    out_ref[...] = _layernorm(u, n_g_ref[...], n_be_ref[...])


def _stage_c(d1, g1, ep, mu_b1, mu_g, mu_be, mu_w2, mu_b2,
             mg_b1, mg_w2, mg_b2, n_g, n_be):
    vec = pl.BlockSpec((_H,), lambda n: (0,))
    blk = pl.BlockSpec((_BN, _H), lambda n: (n, 0))
    mat = pl.BlockSpec((_H, _H), lambda n: (0, 0))
    return pl.pallas_call(
        _stage_c_body,
        grid=(_NB,),
        in_specs=[blk, blk, blk, vec, vec, vec, mat, vec, vec, mat, vec,
                  vec, vec],
        out_specs=blk,
        out_shape=jax.ShapeDtypeStruct((_N, _H), F32),
    )(d1, g1, ep, mu_b1, mu_g, mu_be, mu_w2, mu_b2, mg_b1, mg_w2, mg_b2,
      n_g, n_be)


def kernel(pair_embedding, explanation_path, schema_bucket_ids, hop_counts,
           path_source_ids, params):
    p = params
    sid = jnp.clip(schema_bucket_ids, 0, _SCHEMA).astype(jnp.int32)
    hid = jnp.clip(hop_counts, 0, _MAXHOP + 1).astype(jnp.int32)
    pid = jnp.clip(path_source_ids, 0, _SRCVOC - 1).astype(jnp.int32)

    w1 = p['r_w1']
    w_pair = w1[0:_H]
    w_ep = w1[_H:2 * _H]
    w_s = w1[2 * _H:2 * _H + _MD]
    w_h = w1[2 * _H + _MD:2 * _H + 2 * _MD]
    w_p = w1[2 * _H + 2 * _MD:]
    hop_pad = jnp.pad(p['hop_emb'], ((0, 16 - (_MAXHOP + 2)), (0, 0)))

    mds = _sc_gather(p['schema_emb'], sid)

    hid3 = hid.reshape(_NBA, 1, _BNA)
    pid3 = pid.reshape(_NBA, 1, _BNA)
    ctx, pw = _stage_ab(pair_embedding, explanation_path, mds, hid3, pid3,
                        w_pair, w_ep, w_s, w_h, w_p, hop_pad,
                        p['source_emb'], p['r_b1'], p['r_g'], p['r_be'],
                        p['r_w2'], p['r_b2'], p['proto'])

    d1, g1 = _mm1(explanation_path, ctx, p['mu_w1'], p['mg_w1'])

    updated = _stage_c(d1, g1, explanation_path, p['mu_b1'], p['mu_g'],
                       p['mu_be'], p['mu_w2'], p['mu_b2'], p['mg_b1'],
                       p['mg_w2'], p['mg_b2'], p['n_g'], p['n_be'])
    return (updated, ctx, pw)


# speedup vs baseline: 1.2220x; 1.2220x over previous
import functools

import jax
import jax.numpy as jnp
from jax import lax
from jax.experimental import pallas as pl
from jax.experimental.pallas import tpu as pltpu
from jax.experimental.pallas import tpu_sc as plsc

F32 = jnp.float32
BF16 = jnp.bfloat16
_N, _H, _MD, _NP, _RH = 8192, 2048, 512, 64, 2048
_SCHEMA, _MAXHOP, _SRCVOC = 2048, 8, 8
_RS = 0.2
_BN = 256
_NB = _N // _BN
_BJ = 512
_SC_C = 64
_SQRT1_2 = 0.7071067811865476


def _gelu_exact(x):
    return 0.5 * x * (1.0 + lax.erf(x * _SQRT1_2))


def _layernorm(x, g, b, eps=1e-5):
    mu = jnp.mean(x, axis=-1, keepdims=True)
    xc = x - mu
    var = jnp.mean(xc * xc, axis=-1, keepdims=True)
    return xc * lax.rsqrt(var + eps) * g + b


def _sc_gather(table, idx):
    info = plsc.get_sparse_core_info()
    nw = info.num_cores * info.num_subcores
    rows_per_w = _N // nw
    n_chunks = rows_per_w // _SC_C
    mesh = plsc.VectorSubcoreMesh(core_axis_name="c", subcore_axis_name="s")
    md = table.shape[1]

    @functools.partial(
        pl.kernel, mesh=mesh,
        out_type=jax.ShapeDtypeStruct((_N, md), F32),
        scratch_types=[pltpu.VMEM((rows_per_w,), jnp.int32),
                       pltpu.VMEM((_SC_C, md), F32),
                       pltpu.VMEM((_SC_C, md), F32),
                       pltpu.SemaphoreType.DMA],
    )
    def gather_k(table_hbm, idx_hbm, out_hbm, idx_v, buf0, buf1, sem):
        wid = lax.axis_index("s") * info.num_cores + lax.axis_index("c")
        base = wid * rows_per_w
        pltpu.sync_copy(idx_hbm.at[pl.ds(base, rows_per_w)], idx_v)
        bufs = (buf0, buf1)
        cps = [
            pltpu.async_copy(table_hbm.at[idx_v.at[pl.ds(k * _SC_C, _SC_C)]],
                             bufs[k], sem)
            for k in range(2)
        ]
        for k in range(n_chunks):
            cps[k % 2].wait()
            pltpu.sync_copy(bufs[k % 2],
                            out_hbm.at[pl.ds(base + k * _SC_C, _SC_C)])
            if k + 2 < n_chunks:
                cps[k % 2] = pltpu.async_copy(
                    table_hbm.at[idx_v.at[pl.ds((k + 2) * _SC_C, _SC_C)]],
                    bufs[k % 2], sem)

    return gather_k(table, idx)


def _fold_body(hop_ref, src_ref, w_h_ref, w_p_ref, proto_ref,
               wcmu_ref, wcmg_ref, thp_ref, pbmu_ref, pbmg_ref):
    thp_ref[0:16, :] = jnp.dot(hop_ref[...], w_h_ref[...],
                               preferred_element_type=F32)
    thp_ref[16:24, :] = jnp.dot(src_ref[...], w_p_ref[...],
                                preferred_element_type=F32)
    thp_ref[24:32, :] = jnp.zeros((8, _H), F32)
    pbmu_ref[...] = jnp.dot(proto_ref[...], wcmu_ref[...],
                            preferred_element_type=F32)
    pbmg_ref[...] = jnp.dot(proto_ref[...], wcmg_ref[...],
                            preferred_element_type=F32)


def _fold_tables(hop_pad, src, w_h, w_p, proto, wcmu, wcmg):
    return pl.pallas_call(
        _fold_body,
        out_shape=(jax.ShapeDtypeStruct((32, _H), F32),
                   jax.ShapeDtypeStruct((_NP, _H), F32),
                   jax.ShapeDtypeStruct((_NP, _H), F32)),
    )(hop_pad, src, w_h, w_p, proto, wcmu, wcmg)


def _stage_ab_body(pair_ref, ep_ref, mds_ref, hid_ref, pid_ref,
                   w_pair_ref, w_ep_ref, w_s_ref, thp_ref,
                   b1_ref, g1_ref, be1_ref,
                   w2_ref, b2_ref, proto_ref,
                   ctx_ref, pw_ref, epb_ref, ctxb_ref):
    acc = jnp.dot(pair_ref[...], w_pair_ref[...], preferred_element_type=F32)
    acc = acc + jnp.dot(ep_ref[...], w_ep_ref[...], preferred_element_type=F32)
    acc = acc + jnp.dot(mds_ref[...], w_s_ref[...], preferred_element_type=F32)
    hid = hid_ref[0, 0, :]
    pid = pid_ref[0, 0, :]
    cols = lax.broadcasted_iota(jnp.int32, (_BN, 32), 1)
    oh = jnp.where((hid[:, None] == cols) | ((pid[:, None] + 16) == cols),
                   1.0, 0.0).astype(F32)
    acc = acc + jnp.dot(oh, thp_ref[...], preferred_element_type=F32,
                        precision=lax.Precision.HIGHEST)
    acc = acc + b1_ref[...]
    h = _layernorm(acc, g1_ref[...], be1_ref[...])
    h = _gelu_exact(h)
    logits = jnp.dot(h, w2_ref[...], preferred_element_type=F32) + b2_ref[...]
    cur = logits
    t = None
    for _ in range(4):
        t = jnp.max(cur, axis=-1, keepdims=True)
        cur = jnp.where(cur >= t, -jnp.inf, cur)
    mask = logits >= t
    m1 = jnp.max(logits, axis=-1, keepdims=True)
    e = jnp.exp(logits - m1)
    es = jnp.where(mask, e, 0.0)
    pw = es / jnp.sum(es, axis=-1, keepdims=True)
    pw_ref[...] = pw
    ctx = jnp.dot(pw, proto_ref[...], preferred_element_type=F32)
    ctx_ref[...] = ctx
    epb_ref[...] = ep_ref[...].astype(BF16)
    ctxb_ref[...] = ctx.astype(BF16)


def _stage_ab(pair, ep, mds, hid3, pid3, w_pair, w_ep, w_s, thp,
              b1, g1, be1, w2, b2, proto):
    return pl.pallas_call(
        _stage_ab_body,
        grid=(_NB,),
        in_specs=[
            pl.BlockSpec((_BN, _H), lambda n: (n, 0)),
            pl.BlockSpec((_BN, _H), lambda n: (n, 0)),
            pl.BlockSpec((_BN, _MD), lambda n: (n, 0)),
            pl.BlockSpec((1, 1, _BN), lambda n: (n, 0, 0)),
            pl.BlockSpec((1, 1, _BN), lambda n: (n, 0, 0)),
            pl.BlockSpec((_H, _H), lambda n: (0, 0)),
            pl.BlockSpec((_H, _H), lambda n: (0, 0)),
            pl.BlockSpec((_MD, _H), lambda n: (0, 0)),
            pl.BlockSpec((32, _H), lambda n: (0, 0)),
            pl.BlockSpec((_H,), lambda n: (0,)),
            pl.BlockSpec((_H,), lambda n: (0,)),
            pl.BlockSpec((_H,), lambda n: (0,)),
            pl.BlockSpec((_H, _NP), lambda n: (0, 0)),
            pl.BlockSpec((_NP,), lambda n: (0,)),
            pl.BlockSpec((_NP, _H), lambda n: (0, 0)),
        ],
        out_specs=(pl.BlockSpec((_BN, _H), lambda n: (n, 0)),
                   pl.BlockSpec((_BN, _NP), lambda n: (n, 0)),
                   pl.BlockSpec((_BN, _H), lambda n: (n, 0)),
                   pl.BlockSpec((_BN, _H), lambda n: (n, 0))),
        out_shape=(jax.ShapeDtypeStruct((_N, _H), F32),
                   jax.ShapeDtypeStruct((_N, _NP), F32),
                   jax.ShapeDtypeStruct((_N, _H), BF16),
                   jax.ShapeDtypeStruct((_N, _H), BF16)),
    )(pair, ep, mds, hid3, pid3, w_pair, w_ep, w_s, thp,
      b1, g1, be1, w2, b2, proto)


def _mm1_body(epb_ref, ctxb_ref, pw_ref,
              wmu_e_ref, wmu_a_ref, wmu_p_ref,
              wmg_e_ref, wmg_a_ref, wmg_p_ref,
              pbmu_ref, pbmg_ref, d_ref, g_ref):
    ep = epb_ref[...]
    ctx = ctxb_ref[...]
    f2 = jnp.abs(ep - ctx)
    f3 = ep * ctx
    pw = pw_ref[...]
    parts = (
        (wmu_e_ref, wmu_a_ref, wmu_p_ref, pbmu_ref, d_ref),
        (wmg_e_ref, wmg_a_ref, wmg_p_ref, pbmg_ref, g_ref),
    )
    for we, wa, wp_, pb, o_ref in parts:
        acc = jnp.dot(ep, we[...], preferred_element_type=F32)
        acc = acc + jnp.dot(f2, wa[...], preferred_element_type=F32)
        acc = acc + jnp.dot(f3, wp_[...], preferred_element_type=F32)
        acc = acc + jnp.dot(pw, pb[...], preferred_element_type=F32)
        o_ref[...] = acc.astype(BF16)


def _mm1(epb, ctxb, pw, wmu, wmg, pbmu, pbmg):
    nj = _H // _BJ
    wblk = lambda r: pl.BlockSpec((_H, _BJ), lambda j, n, _r=r: (_r, j))
    return pl.pallas_call(
        _mm1_body,
        grid=(nj, _NB),
        in_specs=[
            pl.BlockSpec((_BN, _H), lambda j, n: (n, 0)),
            pl.BlockSpec((_BN, _H), lambda j, n: (n, 0)),
            pl.BlockSpec((_BN, _NP), lambda j, n: (n, 0)),
            wblk(0), wblk(2), wblk(3),
            wblk(0), wblk(2), wblk(3),
            pl.BlockSpec((_NP, _BJ), lambda j, n: (0, j)),
            pl.BlockSpec((_NP, _BJ), lambda j, n: (0, j)),
        ],
        out_specs=(pl.BlockSpec((_BN, _BJ), lambda j, n: (n, j)),
                   pl.BlockSpec((_BN, _BJ), lambda j, n: (n, j))),
        out_shape=(jax.ShapeDtypeStruct((_N, _H), BF16),
                   jax.ShapeDtypeStruct((_N, _H), BF16)),
    )(epb, ctxb, pw, wmu, wmu, wmu, wmg, wmg, wmg, pbmu, pbmg)


def _stage_c_body(d1_ref, g1_ref, ep_ref, mu_b1_ref, mu_g_ref, mu_be_ref,
                  mu_w2_ref, mu_b2_ref, mg_b1_ref, mg_w2_ref, mg_b2_ref,
                  n_g_ref, n_be_ref, out_ref):
    d1 = d1_ref[...].astype(F32) + mu_b1_ref[...]
    d1 = _layernorm(d1, mu_g_ref[...], mu_be_ref[...])
    d1 = _gelu_exact(d1)
    d = jnp.dot(d1, mu_w2_ref[...], preferred_element_type=F32) + mu_b2_ref[...]
    g1 = g1_ref[...].astype(F32) + mg_b1_ref[...]
    g1 = _gelu_exact(g1)
    g = jnp.dot(g1, mg_w2_ref[...], preferred_element_type=F32) + mg_b2_ref[...]
    g = jax.nn.sigmoid(g)
    u = ep_ref[...] + _RS * g * d
    out_ref[...] = _layernorm(u, n_g_ref[...], n_be_ref[...])


def _stage_c(d1, g1, ep, mu_b1, mu_g, mu_be, mu_w2, mu_b2,
             mg_b1, mg_w2, mg_b2, n_g, n_be):
    vec = pl.BlockSpec((_H,), lambda n: (0,))
    blk = pl.BlockSpec((_BN, _H), lambda n: (n, 0))
    mat = pl.BlockSpec((_H, _H), lambda n: (0, 0))
    return pl.pallas_call(
        _stage_c_body,
        grid=(_NB,),
        in_specs=[blk, blk, blk, vec, vec, vec, mat, vec, vec, mat, vec,
                  vec, vec],
        out_specs=blk,
        out_shape=jax.ShapeDtypeStruct((_N, _H), F32),
    )(d1, g1, ep, mu_b1, mu_g, mu_be, mu_w2, mu_b2, mg_b1, mg_w2, mg_b2,
      n_g, n_be)


def kernel(pair_embedding, explanation_path, schema_bucket_ids, hop_counts,
           path_source_ids, params):
    p = params
    sid = jnp.clip(schema_bucket_ids, 0, _SCHEMA).astype(jnp.int32)
    hid = jnp.clip(hop_counts, 0, _MAXHOP + 1).astype(jnp.int32)
    pid = jnp.clip(path_source_ids, 0, _SRCVOC - 1).astype(jnp.int32)

    w1 = p['r_w1']
    w_pair = w1[0:_H]
    w_ep = w1[_H:2 * _H]
    w_s = w1[2 * _H:2 * _H + _MD]
    w_h = w1[2 * _H + _MD:2 * _H + 2 * _MD]
    w_p = w1[2 * _H + 2 * _MD:]
    hop_pad = jnp.pad(p['hop_emb'], ((0, 16 - (_MAXHOP + 2)), (0, 0)))

    mds = _sc_gather(p['schema_emb'], sid)
    thp, pbmu, pbmg = _fold_tables(hop_pad, p['source_emb'], w_h, w_p,
                                   p['proto'], p['mu_w1'][_H:2 * _H],
                                   p['mg_w1'][_H:2 * _H])

    hid3 = hid.reshape(_NB, 1, _BN)
    pid3 = pid.reshape(_NB, 1, _BN)
    ctx, pw, epb, ctxb = _stage_ab(pair_embedding, explanation_path, mds,
                                   hid3, pid3, w_pair, w_ep, w_s, thp,
                                   p['r_b1'], p['r_g'], p['r_be'],
                                   p['r_w2'], p['r_b2'], p['proto'])

    d1, g1 = _mm1(epb, ctxb, pw, p['mu_w1'], p['mg_w1'], pbmu, pbmg)

    updated = _stage_c(d1, g1, explanation_path, p['mu_b1'], p['mu_g'],
                       p['mu_be'], p['mu_w2'], p['mu_b2'], p['mg_b1'],
                       p['mg_w2'], p['mg_b2'], p['n_g'], p['n_be'])
    return (updated, ctx, pw)
